# baseline (speedup 1.0000x reference)
import math

import numpy as np
import jax
import jax.numpy as jnp
from jax.experimental import pallas as pl
from jax.experimental.pallas import tpu as pltpu

OUT_F = 100000
SUB = 3
SCALE = 32.0
MARGIN = 0.2
COS_M = math.cos(MARGIN)
SIN_M = math.sin(MARGIN)
TH = math.cos(math.pi - MARGIN)
MMM = 1.0 + math.cos(math.pi - MARGIN)

BM = 256
BN = 2048
CH = 128
NS = 4
JLAST = (OUT_F + BN - 1) // BN - 1

_S_NP = np.zeros((SUB * CH, CH), dtype=np.float32)
_S_NP[np.arange(CH) * SUB, np.arange(CH)] = 1.0


def _body(lab_ref, s_ref, *refs):
    x_refs = refs[:NS]
    out_ref = refs[NS]
    j = pl.program_id(1)
    x16 = jnp.concatenate([r[...] for r in x_refs], axis=0).astype(jnp.bfloat16)

    def _masked(v):
        pcol = j * SUB * BN + jax.lax.broadcasted_iota(
            jnp.int32, (1, SUB * BN), 1)
        return jnp.where(pcol < SUB * OUT_F, v, jnp.bfloat16(0.0))

    x16 = jax.lax.cond(j == JLAST, _masked, lambda v: v, x16)
    m16 = jnp.maximum(jnp.maximum(x16, jnp.roll(x16, -1, axis=1)),
                      jnp.roll(x16, -2, axis=1))
    s = s_ref[...]
    parts = []
    for t in range(BN // CH):
        chunk = m16[:, t * SUB * CH:(t + 1) * SUB * CH]
        parts.append(
            jax.lax.dot(chunk, s, preferred_element_type=jnp.float32))
    c = jnp.concatenate(parts, axis=1)
    sine = jnp.sqrt(jnp.maximum(1.0 - c * c, 0.0))
    phi = c * COS_M - sine * SIN_M
    phi = jnp.where(c > TH, phi, c - MMM)
    col = j * BN + jax.lax.broadcasted_iota(jnp.int32, (BM, BN), 1)
    mask = lab_ref[...] == col
    out_ref[...] = jnp.where(mask, phi, c) * SCALE


def _slice_spec(q):
    return pl.BlockSpec((BM // NS, SUB * BN), lambda i, j, q=q: (i * NS + q, j))


def kernel(cosine, label):
    B = cosine.shape[0]
    lab2d = label.reshape(B, 1)
    sel = jnp.asarray(_S_NP, dtype=jnp.bfloat16)
    grid = (B // BM, pl.cdiv(OUT_F, BN))
    return pl.pallas_call(
        _body,
        grid=grid,
        in_specs=[
            pl.BlockSpec((BM, 1), lambda i, j: (i, 0)),
            pl.BlockSpec((SUB * CH, CH), lambda i, j: (0, 0)),
        ] + [_slice_spec(q) for q in range(NS)],
        out_specs=pl.BlockSpec((BM, BN), lambda i, j: (i, j)),
        out_shape=jax.ShapeDtypeStruct((B, OUT_F), cosine.dtype),
        compiler_params=pltpu.CompilerParams(
            dimension_semantics=("parallel", "parallel"),
        ),
    )(lab2d, sel, *([cosine] * NS))
